# baseline (speedup 1.0000x reference)
import functools

import jax
import jax.numpy as jnp
from jax import lax
from jax.experimental import pallas as pl
from jax.experimental.pallas import tpu as pltpu
from jax.experimental.pallas import tpu_sc as plsc

DIM = 32
POS_DIM = 16

_NC = 2
_NS = 16
_NW = _NC * _NS

_IDX_W = 128


def _sc_gather(table, idx2d, n_rows):
    rows_per_w = n_rows // _NW
    k_per_w = rows_per_w // _IDX_W
    k_chunk = 20
    n_chunks = k_per_w // k_chunk
    assert k_chunk * n_chunks == k_per_w, (k_per_w, k_chunk)
    c_rows = k_chunk * _IDX_W

    mesh = plsc.VectorSubcoreMesh(core_axis_name="c", subcore_axis_name="s")

    @functools.partial(
        pl.kernel,
        mesh=mesh,
        compiler_params=pltpu.CompilerParams(use_tc_tiling_on_sc=False),
        out_type=jax.ShapeDtypeStruct((n_rows, DIM), jnp.float32),
        scratch_types=[
            pltpu.VMEM((k_per_w, _IDX_W), jnp.int32),
            pltpu.VMEM((c_rows, DIM), jnp.float32),
            pltpu.SemaphoreType.DMA,
        ],
    )
    def gather_kernel(table_hbm, idx_hbm, out_hbm, idx_v, rows_v, sem):
        wid = lax.axis_index("s") * _NC + lax.axis_index("c")
        grp_base = wid * k_per_w
        pltpu.sync_copy(idx_hbm.at[pl.ds(grp_base, k_per_w)], idx_v)

        def chunk_body(it, carry):
            g0 = it * k_chunk
            copies = []
            for k in range(k_chunk):
                copies.append(
                    pltpu.async_copy(
                        table_hbm.at[idx_v.at[g0 + k]],
                        rows_v.at[pl.ds(k * _IDX_W, _IDX_W)],
                        sem,
                    )
                )
            for c in copies:
                c.wait()
            pltpu.sync_copy(
                rows_v,
                out_hbm.at[pl.ds((grp_base + g0) * _IDX_W, c_rows)])
            return carry

        lax.fori_loop(0, n_chunks, chunk_body, 0)

    return gather_kernel(table, idx2d)


def _mlp_concat_kernel(emb_ref, pos_ref, w1_ref, b1_ref, w2_ref, b2_ref,
                       out_ref):
    h = jnp.maximum(
        jnp.dot(pos_ref[...], w1_ref[...],
                preferred_element_type=jnp.float32) + b1_ref[...], 0.0)
    p = jnp.maximum(
        jnp.dot(h, w2_ref[...],
                preferred_element_type=jnp.float32) + b2_ref[...], 0.0)
    out_ref[:, 0:DIM] = emb_ref[...]
    out_ref[:, DIM:2 * DIM] = p


def _tc_mlp_concat(emb, posf, W1, b1, W2, b2, n_rows):
    rows_blk = 8192
    grid = (n_rows // rows_blk,)
    return pl.pallas_call(
        _mlp_concat_kernel,
        grid=grid,
        in_specs=[
            pl.BlockSpec((rows_blk, DIM), lambda i: (i, 0)),
            pl.BlockSpec((rows_blk, POS_DIM), lambda i: (i, 0)),
            pl.BlockSpec((POS_DIM, POS_DIM), lambda i: (0, 0)),
            pl.BlockSpec((1, POS_DIM), lambda i: (0, 0)),
            pl.BlockSpec((POS_DIM, DIM), lambda i: (0, 0)),
            pl.BlockSpec((1, DIM), lambda i: (0, 0)),
        ],
        out_specs=pl.BlockSpec((rows_blk, 2 * DIM), lambda i: (i, 0)),
        out_shape=jax.ShapeDtypeStruct((n_rows, 2 * DIM), jnp.float32),
    )(emb, posf, W1, b1.reshape(1, -1), W2, b2.reshape(1, -1))


def kernel(words, positions, table, W1, b1, W2, b2):
    B, L = words.shape
    n_rows = B * L
    idx2d = words.reshape(n_rows // _IDX_W, _IDX_W)
    emb = _sc_gather(table, idx2d, n_rows)
    posf = positions.reshape(n_rows, POS_DIM)
    out = _tc_mlp_concat(emb, posf, W1, b1, W2, b2, n_rows)
    return out.reshape(B, L, 2 * DIM)


# speedup vs baseline: 1.9305x; 1.9305x over previous
import functools

import jax
import jax.numpy as jnp
from jax import lax
from jax.experimental import pallas as pl
from jax.experimental.pallas import tpu as pltpu
from jax.experimental.pallas import tpu_sc as plsc

DIM = 32
POS_DIM = 16

_NC = 2
_NS = 16
_NW = _NC * _NS

_IDX_W = 128

_T_BLK = 1024
_K_UNIT = _T_BLK // _IDX_W


def _sc_gather(table, idx2d, n_rows):
    rows_per_w = n_rows // _NW
    k_per_w = rows_per_w // _IDX_W
    n_chunks = k_per_w // _K_UNIT
    assert n_chunks * _K_UNIT == k_per_w

    mesh = plsc.VectorSubcoreMesh(core_axis_name="c", subcore_axis_name="s")

    @functools.partial(
        pl.kernel,
        mesh=mesh,
        compiler_params=pltpu.CompilerParams(use_tc_tiling_on_sc=False,
                                             needs_layout_passes=False),
        out_type=jax.ShapeDtypeStruct((n_rows, DIM), jnp.float32),
        scratch_types=[
            pltpu.VMEM((k_per_w, _IDX_W), jnp.int32),
            pltpu.VMEM((_K_UNIT, _IDX_W), jnp.int32),
            pltpu.VMEM((_T_BLK, DIM), jnp.float32),
            pltpu.SemaphoreType.DMA,
        ],
    )
    def gather_kernel(table_hbm, idx_hbm, out_hbm, idx_v, pidx_v, rows_v,
                      sem):
        wid = lax.axis_index("s") * _NC + lax.axis_index("c")
        grp_base = wid * k_per_w
        pltpu.sync_copy(idx_hbm.at[pl.ds(grp_base, k_per_w)], idx_v)

        u = lax.iota(jnp.int32, 16)
        kbase = (u & 3) * (_T_BLK // 4) + (u >> 2)

        def chunk_body(it, carry):
            g0 = it * _K_UNIT
            for g in range(_K_UNIT):
                for v in range(8):
                    kv = kbase + (g * 32 + 4 * v)
                    row = g0 + (kv >> 7)
                    col = kv & (_IDX_W - 1)
                    pidx_v[g, pl.ds(16 * v, 16)] = plsc.load_gather(
                        idx_v, [row, col])
            copies = []
            for g in range(_K_UNIT):
                copies.append(
                    pltpu.async_copy(
                        table_hbm.at[pidx_v.at[g]],
                        rows_v.at[pl.ds(g * _IDX_W, _IDX_W)],
                        sem,
                    )
                )
            for c in copies:
                c.wait()
            pltpu.sync_copy(
                rows_v,
                out_hbm.at[pl.ds((grp_base + g0) * _IDX_W, _T_BLK)])
            return carry

        lax.fori_loop(0, n_chunks, chunk_body, 0)

    return gather_kernel(table, idx2d)


def _mlp_concat_kernel(emb_ref, pos_ref, w1t_ref, b1_ref, w2t_ref, b2_ref,
                       out_ref):
    x = pos_ref[0]
    h = jnp.maximum(
        jnp.dot(w1t_ref[...], x, preferred_element_type=jnp.float32)
        + b1_ref[...], 0.0)
    p = jnp.maximum(
        jnp.dot(w2t_ref[...], h, preferred_element_type=jnp.float32)
        + b2_ref[...], 0.0)
    ec = emb_ref[0, 0]
    ect = ec.T
    et = jnp.concatenate(
        [ect[0:DIM], ect[DIM:2 * DIM], ect[2 * DIM:3 * DIM],
         ect[3 * DIM:4 * DIM]], axis=1)
    out_ref[0, 0:DIM, :] = et
    out_ref[0, DIM:2 * DIM, :] = p


def _tc_mlp_concat(emb3, pos3, W1, b1, W2, b2, B, L):
    grid = (L, B // _T_BLK)
    return pl.pallas_call(
        _mlp_concat_kernel,
        grid=grid,
        in_specs=[
            pl.BlockSpec((1, 1, _T_BLK // 4, 128), lambda l, c: (l, c, 0, 0)),
            pl.BlockSpec((1, POS_DIM, _T_BLK), lambda l, c: (l, 0, c)),
            pl.BlockSpec((POS_DIM, POS_DIM), lambda l, c: (0, 0)),
            pl.BlockSpec((POS_DIM, 1), lambda l, c: (0, 0)),
            pl.BlockSpec((DIM, POS_DIM), lambda l, c: (0, 0)),
            pl.BlockSpec((DIM, 1), lambda l, c: (0, 0)),
        ],
        out_specs=pl.BlockSpec((1, 2 * DIM, _T_BLK), lambda l, c: (l, 0, c)),
        out_shape=jax.ShapeDtypeStruct((L, 2 * DIM, B), jnp.float32),
    )(emb3, pos3, W1.T, b1.reshape(POS_DIM, 1), W2.T, b2.reshape(DIM, 1))


def kernel(words, positions, table, W1, b1, W2, b2):
    B, L = words.shape
    n_rows = B * L
    idx2d = words.T.reshape(n_rows // _IDX_W, _IDX_W)
    emb = _sc_gather(table, idx2d, n_rows)
    emb3 = emb.reshape(L, B // _T_BLK, _T_BLK // 4, 128)
    pos3 = positions.transpose(1, 2, 0)
    out_fm = _tc_mlp_concat(emb3, pos3, W1, b1, W2, b2, B, L)
    return out_fm.transpose(2, 0, 1)
